# Initial kernel scaffold; baseline (speedup 1.0000x reference)
import jax
import jax.numpy as jnp
from jax.experimental import pallas as pl


def kernel(node_feat_idx, degree, edge_feat_idx, adj, position_bias, atom_table, edge_table, degree_table, node_vnode, node_vnode_distance, diffusion_weight):
    raise NotImplementedError("write your pallas kernel here")


# speedup vs baseline: 5.7255x; 5.7255x over previous
import functools

import jax
import jax.numpy as jnp
from jax import lax
from jax.experimental import pallas as pl
from jax.experimental.pallas import tpu as pltpu
from jax.experimental.pallas import tpu_sc as plsc

B, N, H, D = 128, 64, 32, 768
NUM_HOPS = 5
ATOM_VOCAB = 512 * 9 + 1
EDGE_VOCAB = 512 * 3 + 1
DEG_VOCAB = 512
VCOMB = ATOM_VOCAB + DEG_VOCAB

NPOS = N * N
NCHUNK = 8
CHUNK = NPOS // NCHUNK

NC, NS = 2, 16
NW = NC * NS

BB = 8


def _apow_body(adj_ref, apow_ref, explored_ref):
    a32 = adj_ref[...]
    a16 = a32.astype(jnp.bfloat16)
    ak = a16
    acc = a32
    for hop in range(NUM_HOPS):
        apow_ref[:, hop] = ak.astype(jnp.float32)
        prod = lax.dot_general(
            ak, a16,
            dimension_numbers=(((2,), (1,)), ((0,), (0,))),
            preferred_element_type=jnp.float32)
        akn = jnp.minimum(prod, 1.0)
        acc = acc + akn
        ak = akn.astype(jnp.bfloat16)
    explored_ref[...] = (acc > 0).astype(jnp.float32)


_apow_call = pl.pallas_call(
    _apow_body,
    grid=(B // BB,),
    in_specs=[pl.BlockSpec((BB, N, N), lambda i: (i, 0, 0))],
    out_specs=[
        pl.BlockSpec((BB, NUM_HOPS, N, N), lambda i: (i, 0, 0, 0)),
        pl.BlockSpec((BB, N, N), lambda i: (i, 0, 0)),
    ],
    out_shape=[
        jax.ShapeDtypeStruct((B, NUM_HOPS, N, N), jnp.float32),
        jax.ShapeDtypeStruct((B, N, N), jnp.float32),
    ],
)

PAIRS = B * N
PPW = PAIRS // NW
CP = 4
ROWS = CP * 10
NCHN = PPW // CP

_sc_mesh = plsc.VectorSubcoreMesh(core_axis_name="c", subcore_axis_name="s")


@functools.partial(
    pl.kernel,
    mesh=_sc_mesh,
    out_type=jax.ShapeDtypeStruct((PAIRS, D), jnp.float32),
    compiler_params=pltpu.CompilerParams(needs_layout_passes=False, use_tc_tiling_on_sc=False),
    scratch_types=[
        pltpu.VMEM((NCHN, ROWS), jnp.int32),
        pltpu.VMEM((ROWS, D), jnp.float32),
        pltpu.VMEM((CP, D), jnp.float32),
        pltpu.SemaphoreType.DMA,
    ],
)
def _node_gather(table_hbm, idx_hbm, out_hbm, idx_v, rows_v, out_v, sem):
    wid = lax.axis_index("s") * NC + lax.axis_index("c")
    pltpu.sync_copy(idx_hbm.at[wid], idx_v)

    def chunk(c, carry):
        pltpu.async_copy(table_hbm.at[idx_v.at[c]], rows_v, sem).wait()
        for p in range(CP):
            def dloop(j, carry2):
                sl = pl.ds(j * 16, 16)
                acc = rows_v[p * 10, sl]
                for k in range(1, 10):
                    acc = acc + rows_v[p * 10 + k, sl]
                out_v[p, sl] = acc
                return carry2
            lax.fori_loop(0, D // 16, dloop, 0)
        pltpu.sync_copy(out_v, out_hbm.at[pl.ds(wid * PPW + c * CP, CP), :])
        return carry

    lax.fori_loop(0, NCHN, chunk, 0)


BPW = B // NW


@functools.partial(
    pl.kernel,
    mesh=_sc_mesh,
    out_type=jax.ShapeDtypeStruct((B, H, NPOS), jnp.float32),
    compiler_params=pltpu.CompilerParams(needs_layout_passes=False, use_tc_tiling_on_sc=False),
    scratch_types=[
        pltpu.VMEM((EDGE_VOCAB, H), jnp.float32),
        pltpu.VMEM((3, CHUNK), jnp.int32),
        pltpu.VMEM((NUM_HOPS, CHUNK), jnp.float32),
        pltpu.VMEM((H, CHUNK), jnp.float32),
        pltpu.VMEM((H, CHUNK), jnp.float32),
        pltpu.VMEM((NUM_HOPS, H, 16), jnp.float32),
        pltpu.SemaphoreType.DMA,
    ],
)
def _edge_merge(tab_hbm, eidx_hbm, pos_hbm, apow_hbm, w_hbm, att_hbm,
                tab_v, eidx_v, apow_v, pos_v, out_v, w_v, sem):
    wid = lax.axis_index("s") * NC + lax.axis_index("c")
    pltpu.sync_copy(tab_hbm, tab_v)
    pltpu.sync_copy(w_hbm, w_v)

    def body(t, carry):
        b = wid * BPW + t // NCHUNK
        c = t % NCHUNK
        pltpu.sync_copy(eidx_hbm.at[b, c], eidx_v)
        pltpu.sync_copy(apow_hbm.at[b, c], apow_v)
        pltpu.sync_copy(pos_hbm.at[b, :, pl.ds(c * CHUNK, CHUNK)], pos_v)

        def group(g, carry2):
            sl = pl.ds(g * 16, 16)
            e0 = eidx_v[0, sl]
            e1 = eidx_v[1, sl]
            e2 = eidx_v[2, sl]
            a = [apow_v[hop, sl] for hop in range(NUM_HOPS)]
            for h in range(H):
                hsplat = jnp.full((16,), h, jnp.int32)
                fac = jnp.ones((16,), jnp.float32)
                for hop in range(NUM_HOPS):
                    fac = fac + w_v[hop, h] * a[hop]
                g0 = plsc.load_gather(tab_v, [e0, hsplat])
                g1 = plsc.load_gather(tab_v, [e1, hsplat])
                g2 = plsc.load_gather(tab_v, [e2, hsplat])
                out_v[h, sl] = (pos_v[h, sl] + g0 + g1 + g2) * fac
            return carry2

        lax.fori_loop(0, CHUNK // 16, group, 0)
        pltpu.sync_copy(out_v, att_hbm.at[b, :, pl.ds(c * CHUNK, CHUNK)])
        return carry

    lax.fori_loop(0, BPW * NCHUNK, body, 0)


def kernel(node_feat_idx, degree, edge_feat_idx, adj, position_bias,
           atom_table, edge_table, degree_table, node_vnode,
           node_vnode_distance, diffusion_weight):
    combined = jnp.concatenate([atom_table, degree_table], axis=0)
    idx_all = jnp.concatenate(
        [node_feat_idx, degree[..., None] + ATOM_VOCAB], axis=-1)
    idx_node = idx_all.astype(jnp.int32).reshape(NW, NCHN, ROWS)
    node_features = _node_gather(combined, idx_node).reshape(B, N, D)

    apow, explored = _apow_call(adj)
    apow4 = apow.reshape(B, NUM_HOPS, NCHUNK, CHUNK).transpose(0, 2, 1, 3)
    eidx4 = edge_feat_idx.astype(jnp.int32).reshape(
        B, NCHUNK, CHUNK, 3).transpose(0, 1, 3, 2)
    pos3 = position_bias.reshape(B, H, NPOS)
    wexp = jnp.broadcast_to(
        diffusion_weight[:, :, None], (NUM_HOPS, H, 16))
    att3 = _edge_merge(edge_table, eidx4, pos3, apow4, wexp)
    att_bias = att3.reshape(B, H, N, N)
    return (node_features, att_bias, explored, node_vnode,
            node_vnode_distance)
